# Initial kernel scaffold; baseline (speedup 1.0000x reference)
import jax
import jax.numpy as jnp
from jax.experimental import pallas as pl


def kernel(x, edge_index, batch, W1, b1, W2, b2, W3, b3):
    raise NotImplementedError("write your pallas kernel here")


# speedup vs baseline: 4.6055x; 4.6055x over previous
import functools

import jax
import jax.numpy as jnp
from jax import lax
from jax.experimental import pallas as pl
from jax.experimental.pallas import tpu as pltpu
from jax.experimental.pallas import tpu_sc as plsc

N = 10000
E = 320000
D = 128
G = 64
NC = 2
NS = 16
NW = NC * NS
N_PAD = 10240
GP = 128

_MESH = dict(core_axis_name="c", subcore_axis_name="s",
             num_cores=NC, num_subcores=NS)

EPT = E // NW
ECH = 80
ZR = 16


def _fill_zeros(zbuf, nrows):
    zv = jnp.zeros((16,), jnp.float32)
    for i in range(nrows):
        for j in range(D // 16):
            zbuf[i, pl.ds(j * 16, 16)] = zv


def _make_agg():
    mesh = plsc.VectorSubcoreMesh(**_MESH)
    scratch = [
        pltpu.VMEM_SHARED((N_PAD, D), jnp.float32),
        pltpu.VMEM((ECH, D), jnp.float32),
        pltpu.VMEM((ECH,), jnp.int32),
        pltpu.VMEM((ECH,), jnp.int32),
        pltpu.VMEM((ZR, D), jnp.float32),
        pltpu.SemaphoreType.DMA,
    ]

    @functools.partial(
        pl.kernel,
        out_type=jax.ShapeDtypeStruct((NC, N_PAD, D), jnp.float32),
        mesh=mesh, scratch_types=scratch)
    def agg(h_hbm, src_hbm, dst_hbm, out_hbm, acc, rows, sidx, didx, zbuf, sem):
        c = lax.axis_index("c")
        s = lax.axis_index("s")
        _fill_zeros(zbuf, ZR)
        rpt = N_PAD // NS
        for i in range(rpt // ZR):
            pltpu.sync_copy(zbuf, acc.at[pl.ds(s * rpt + i * ZR, ZR), :])
        plsc.subcore_barrier()

        ebase = (c * NS + s) * EPT

        def body(j, carry):
            b = ebase + j * ECH
            pltpu.sync_copy(src_hbm.at[pl.ds(b, ECH)], sidx)
            pltpu.sync_copy(dst_hbm.at[pl.ds(b, ECH)], didx)
            pltpu.async_copy(h_hbm.at[sidx], rows, sem).wait()
            pltpu.sync_copy(rows, acc.at[didx], add=True)
            return carry

        lax.fori_loop(0, EPT // ECH, body, 0)
        plsc.subcore_barrier()
        pltpu.sync_copy(acc.at[pl.ds(s * rpt, rpt), :],
                        out_hbm.at[c, pl.ds(s * rpt, rpt), :])

    return agg


def _make_pool():
    mesh = plsc.VectorSubcoreMesh(**_MESH)
    NPT = N_PAD // NW
    CHP = 80
    scratch = [
        pltpu.VMEM_SHARED((GP, D), jnp.float32),
        pltpu.VMEM((CHP, D), jnp.float32),
        pltpu.VMEM((CHP,), jnp.int32),
        pltpu.VMEM((GP // NS, D), jnp.float32),
        pltpu.SemaphoreType.DMA,
    ]

    @functools.partial(
        pl.kernel,
        out_type=jax.ShapeDtypeStruct((NC, GP, D), jnp.float32),
        mesh=mesh, scratch_types=scratch)
    def pool(h_hbm, b_hbm, out_hbm, acc, rows, bidx, zbuf, sem):
        c = lax.axis_index("c")
        s = lax.axis_index("s")
        rpt = GP // NS
        _fill_zeros(zbuf, rpt)
        pltpu.sync_copy(zbuf, acc.at[pl.ds(s * rpt, rpt), :])
        plsc.subcore_barrier()

        nbase = (c * NS + s) * NPT
        for j in range(NPT // CHP):
            b = nbase + j * CHP
            pltpu.sync_copy(b_hbm.at[pl.ds(b, CHP)], bidx)
            pltpu.sync_copy(h_hbm.at[pl.ds(b, CHP), :], rows)
            pltpu.sync_copy(rows, acc.at[bidx], add=True)
        plsc.subcore_barrier()
        pltpu.sync_copy(acc.at[pl.ds(s * rpt, rpt), :],
                        out_hbm.at[c, pl.ds(s * rpt, rpt), :])

    return pool


_agg = _make_agg()
_pool = _make_pool()


def _mm_body(h_ref, p0_ref, p1_ref, w_ref, b_ref, o_ref, *, relu):
    z = h_ref[...] + p0_ref[0] + p1_ref[0]
    y = jnp.dot(z, w_ref[...], preferred_element_type=jnp.float32) + b_ref[...]
    if relu:
        y = jnp.maximum(y, 0.0)
    o_ref[...] = y


def _mm(h, P, w, b, relu):
    BLK = 2048
    return pl.pallas_call(
        functools.partial(_mm_body, relu=relu),
        grid=(N_PAD // BLK,),
        in_specs=[
            pl.BlockSpec((BLK, D), lambda i: (i, 0)),
            pl.BlockSpec((1, BLK, D), lambda i: (0, i, 0)),
            pl.BlockSpec((1, BLK, D), lambda i: (1, i, 0)),
            pl.BlockSpec((D, D), lambda i: (0, 0)),
            pl.BlockSpec((1, D), lambda i: (0, 0)),
        ],
        out_specs=pl.BlockSpec((BLK, D), lambda i: (i, 0)),
        out_shape=jax.ShapeDtypeStruct((N_PAD, D), jnp.float32),
    )(h, P, P, w, b.reshape(1, D))


def _combine_body(p_ref, o_ref):
    o_ref[...] = p_ref[0, :G, :] + p_ref[1, :G, :]


def _combine(Ppool):
    return pl.pallas_call(
        _combine_body,
        out_shape=jax.ShapeDtypeStruct((G, D), jnp.float32),
    )(Ppool)


def kernel(x, edge_index, batch, W1, b1, W2, b2, W3, b3):
    src = edge_index[0]
    dst = edge_index[1]
    h = jnp.pad(x, ((0, N_PAD - N), (0, 0)))
    batch_p = jnp.pad(batch, (0, N_PAD - N), constant_values=G)

    P = _agg(h, src, dst)
    h = _mm(h, P, W1, b1, relu=True)
    P = _agg(h, src, dst)
    h = _mm(h, P, W2, b2, relu=True)
    P = _agg(h, src, dst)
    h = _mm(h, P, W3, b3, relu=False)
    Pp = _pool(h, batch_p)
    return _combine(Pp).reshape(-1)
